# Initial kernel scaffold; baseline (speedup 1.0000x reference)
import jax
import jax.numpy as jnp
from jax.experimental import pallas as pl


def kernel(card_ids, edge_index, edge_attr, batch, emb, Wp, bp, W1, b1, W2, b2, We1, be1, We2, be2, Wf, bf):
    raise NotImplementedError("write your pallas kernel here")


# speedup vs baseline: 1.4718x; 1.4718x over previous
import functools

import jax
import jax.numpy as jnp
from jax import lax
from jax.experimental import pallas as pl
from jax.experimental.pallas import tpu as pltpu
import jax.experimental.pallas.tpu_sc as plsc

N = 50000
E = 800000
G = 1024
HID = 64
NB = 50
BN = N // NB
NH = N // 2
HROWS = 25088
TPH = HROWS // 16
EPT = E // 16
CH = 80
NCH = EPT // CH
PCH = 80
PNCH = N // PCH


def _embed_body(ids_ref, emb_ref, wp_ref, bp_ref, out_ref):
    tbl = jnp.dot(emb_ref[...], wp_ref[...],
                  preferred_element_type=jnp.float32) + bp_ref[...]
    ids = ids_ref[0, 0, :].reshape(BN, 1)
    onehot = (ids == lax.broadcasted_iota(jnp.int32, (BN, 64), 1)
              ).astype(jnp.float32)
    out_ref[...] = jnp.dot(onehot, tbl, preferred_element_type=jnp.float32)


def _mlp_body(x_ref, agg_ref, w1_ref, b1_ref, w2_ref, b2_ref, out_ref):
    u = x_ref[...] + agg_ref[0]
    t = jnp.maximum(
        jnp.dot(u, w1_ref[...], preferred_element_type=jnp.float32)
        + b1_ref[...], 0.0)
    h = jnp.dot(t, w2_ref[...], preferred_element_type=jnp.float32) + b2_ref[...]
    out_ref[...] = jnp.maximum(h, 0.0)


def _final_body(x_ref, agg_ref, w1_ref, b1_ref, w2_ref, b2_ref, wf_ref, bf_ref,
                out_ref):
    u = x_ref[...] + agg_ref[0]
    t = jnp.maximum(
        jnp.dot(u, w1_ref[...], preferred_element_type=jnp.float32)
        + b1_ref[...], 0.0)
    h = jnp.dot(t, w2_ref[...], preferred_element_type=jnp.float32) + b2_ref[...]
    y = jnp.dot(h, wf_ref[...], preferred_element_type=jnp.float32) + bf_ref[...]
    out_ref[:, :64] = y
    ext = (lax.broadcasted_iota(jnp.int32, (BN, 32), 1) == 0).astype(jnp.float32)
    out_ref[:, 64:] = ext


def _divide_body(p_ref, out_ref):
    p = p_ref[...]
    s = p[0] + p[1]
    cnt = jnp.maximum(s[:, 64:65], 1.0)
    out_ref[...] = s[:, :64] / cnt


_MESH = plsc.VectorSubcoreMesh(core_axis_name="c", subcore_axis_name="s",
                               num_cores=2, num_subcores=16)


def _edge_sc_body(x_hbm, src_hbm, dst_hbm, attr_hbm, w_hbm, agg_hbm,
                  agg_sp, wv, srcv, dstv, attrv, xrows, mrows, zbuf, sem):
    c = lax.axis_index("c")
    s = lax.axis_index("s")
    pltpu.sync_copy(w_hbm, wv)
    wk = [[wv[k, pl.ds(g * 16, 16)] for g in range(4)] for k in range(4)]
    bk = [wv[4, pl.ds(g * 16, 16)] for g in range(4)]

    z = jnp.zeros((16,), jnp.float32)

    def zrow(i, carry):
        for g in range(4):
            zbuf[i, pl.ds(g * 16, 16)] = z
        return carry

    lax.fori_loop(0, 224, zrow, 0)

    def zcopy(k, carry):
        pltpu.sync_copy(zbuf, agg_sp.at[pl.ds(s * TPH + k * 224, 224)])
        return carry

    lax.fori_loop(0, 7, zcopy, 0)
    plsc.subcore_barrier()

    base0 = s * EPT
    off = c * NH

    def chunk(i, carry):
        base = base0 + i * CH
        pltpu.sync_copy(src_hbm.at[pl.ds(base, CH)], srcv)
        pltpu.sync_copy(dst_hbm.at[pl.ds(base, CH)], dstv)
        pltpu.sync_copy(attr_hbm.at[pl.ds(base * 4, CH * 4)], attrv)
        pltpu.async_copy(x_hbm.at[srcv], xrows, sem).wait()

        def grp(q, ecarry):
            av = attrv[pl.ds(q * 16, 16)]
            for t in range(4):
                j = q * 4 + t
                a = [av[4 * t + k] for k in range(4)]
                for g in range(4):
                    e = (bk[g] + a[0] * wk[0][g] + a[1] * wk[1][g]
                         + a[2] * wk[2][g] + a[3] * wk[3][g])
                    m = jnp.maximum(xrows[j, pl.ds(g * 16, 16)] + e, 0.0)
                    mrows[j, pl.ds(g * 16, 16)] = m
            return ecarry

        lax.fori_loop(0, CH // 4, grp, 0)

        for v in range(CH // 16):
            d = dstv[pl.ds(v * 16, 16)]
            dl = d - off
            ok = (dl >= 0) & (dl < NH)
            dummy = NH + jnp.bitwise_and(d, 63)
            dstv[pl.ds(v * 16, 16)] = jnp.where(ok, dl, dummy)

        pltpu.sync_copy(mrows, agg_sp.at[dstv], add=True)
        return carry

    lax.fori_loop(0, NCH, chunk, 0)
    plsc.subcore_barrier()
    pltpu.sync_copy(agg_sp.at[pl.ds(s * TPH, TPH)],
                    agg_hbm.at[c, pl.ds(s * TPH, TPH)])


_edge_sc = pl.kernel(
    _edge_sc_body,
    out_type=jax.ShapeDtypeStruct((2, HROWS, HID), jnp.float32),
    mesh=_MESH,
    compiler_params=pltpu.CompilerParams(use_tc_tiling_on_sc=False),
    scratch_types=[
        pltpu.VMEM_SHARED((HROWS, HID), jnp.float32),
        pltpu.VMEM((5, HID), jnp.float32),
        pltpu.VMEM((CH,), jnp.int32),
        pltpu.VMEM((CH,), jnp.int32),
        pltpu.VMEM((CH * 4,), jnp.float32),
        pltpu.VMEM((CH, HID), jnp.float32),
        pltpu.VMEM((CH, HID), jnp.float32),
        pltpu.VMEM((224, HID), jnp.float32),
        pltpu.SemaphoreType.DMA,
    ],
)


def _pool_sc_body(y_hbm, batch_hbm, out_hbm, pool_sp, rows_v, bidv, zbuf):
    c = lax.axis_index("c")
    s = lax.axis_index("s")
    w = c * 16 + s

    z = jnp.zeros((16,), jnp.float32)

    def zrow(i, carry):
        for g in range(6):
            zbuf[i, pl.ds(g * 16, 16)] = z
        return carry

    lax.fori_loop(0, 64, zrow, 0)
    pltpu.sync_copy(zbuf, pool_sp.at[pl.ds(s * 64, 64)])
    plsc.subcore_barrier()

    nch_w = (PNCH - w + 31) // 32

    def chunk(k, carry):
        base = (w + k * 32) * PCH
        pltpu.sync_copy(y_hbm.at[pl.ds(base, PCH)], rows_v)
        pltpu.sync_copy(batch_hbm.at[pl.ds(base, PCH)], bidv)
        pltpu.sync_copy(rows_v, pool_sp.at[bidv], add=True)
        return carry

    lax.fori_loop(0, nch_w, chunk, 0)
    plsc.subcore_barrier()
    pltpu.sync_copy(pool_sp.at[pl.ds(s * 64, 64)],
                    out_hbm.at[c, pl.ds(s * 64, 64)])


_pool_sc = pl.kernel(
    _pool_sc_body,
    out_type=jax.ShapeDtypeStruct((2, G, 96), jnp.float32),
    mesh=_MESH,
    compiler_params=pltpu.CompilerParams(use_tc_tiling_on_sc=False),
    scratch_types=[
        pltpu.VMEM_SHARED((G, 96), jnp.float32),
        pltpu.VMEM((PCH, 96), jnp.float32),
        pltpu.VMEM((PCH,), jnp.int32),
        pltpu.VMEM((64, 96), jnp.float32),
    ],
)


def _agg_spec():
    return pl.BlockSpec((1, BN, HID), lambda i: (i // 25, i % 25, 0))


def _full(shape):
    return pl.BlockSpec(shape, lambda i: tuple(0 for _ in shape))


def kernel(card_ids, edge_index, edge_attr, batch, emb, Wp, bp, W1, b1, W2, b2,
           We1, be1, We2, be2, Wf, bf):
    ids3 = card_ids.reshape(NB, 1, BN)
    emb_pad = jnp.zeros((64, 16), jnp.float32).at[:53, :].set(emb)
    b1r = b1.reshape(1, HID)
    b2r = b2.reshape(1, HID)
    bpr = bp.reshape(1, HID)
    bfr = bf.reshape(1, HID)
    wrow1 = jnp.concatenate([We1, be1[None, :]], axis=0)
    wrow2 = jnp.concatenate([We2, be2[None, :]], axis=0)

    x0 = pl.pallas_call(
        _embed_body,
        grid=(NB,),
        in_specs=[
            pl.BlockSpec((1, 1, BN), lambda i: (i, 0, 0)),
            _full((64, 16)),
            _full((16, HID)),
            _full((1, HID)),
        ],
        out_specs=pl.BlockSpec((BN, HID), lambda i: (i, 0)),
        out_shape=jax.ShapeDtypeStruct((N, HID), jnp.float32),
    )(ids3, emb_pad, Wp, bpr)

    attr_flat = edge_attr.reshape(E * 4)
    src = edge_index[0]
    dst = edge_index[1]

    agg1 = _edge_sc(x0, src, dst, attr_flat, wrow1)

    h1 = pl.pallas_call(
        _mlp_body,
        grid=(NB,),
        in_specs=[
            pl.BlockSpec((BN, HID), lambda i: (i, 0)),
            _agg_spec(),
            _full((HID, HID)),
            _full((1, HID)),
            _full((HID, HID)),
            _full((1, HID)),
        ],
        out_specs=pl.BlockSpec((BN, HID), lambda i: (i, 0)),
        out_shape=jax.ShapeDtypeStruct((N, HID), jnp.float32),
    )(x0, agg1, W1, b1r, W2, b2r)

    agg2 = _edge_sc(h1, src, dst, attr_flat, wrow2)

    yext = pl.pallas_call(
        _final_body,
        grid=(NB,),
        in_specs=[
            pl.BlockSpec((BN, HID), lambda i: (i, 0)),
            _agg_spec(),
            _full((HID, HID)),
            _full((1, HID)),
            _full((HID, HID)),
            _full((1, HID)),
            _full((HID, HID)),
            _full((1, HID)),
        ],
        out_specs=pl.BlockSpec((BN, 96), lambda i: (i, 0)),
        out_shape=jax.ShapeDtypeStruct((N, 96), jnp.float32),
    )(h1, agg2, W1, b1r, W2, b2r, Wf, bfr)

    pooled = _pool_sc(yext, batch)

    out = pl.pallas_call(
        _divide_body,
        in_specs=[pl.BlockSpec((2, G, 96), lambda: (0, 0, 0))],
        out_specs=pl.BlockSpec((G, HID), lambda: (0, 0)),
        out_shape=jax.ShapeDtypeStruct((G, HID), jnp.float32),
    )(pooled)
    return out


# speedup vs baseline: 2.3600x; 1.6035x over previous
import functools

import jax
import jax.numpy as jnp
from jax import lax
from jax.experimental import pallas as pl
from jax.experimental.pallas import tpu as pltpu
import jax.experimental.pallas.tpu_sc as plsc

N = 50000
NP = 50176
E = 800000
G = 1024
GP = 1040
HID = 64
NB = 32
BN = NP // NB
NH = NP // 2
AROWS = NP
TPN = NP // 16
CH = 64
NCHT = E // CH
PCH = 112
PNW = NP // PCH // 32


def _embed_body(ids_ref, emb_ref, wp_ref, bp_ref, out_ref):
    tbl = jnp.dot(emb_ref[...], wp_ref[...],
                  preferred_element_type=jnp.float32) + bp_ref[...]
    ids = ids_ref[0, 0, :].reshape(BN, 1)
    onehot = (ids == lax.broadcasted_iota(jnp.int32, (BN, 64), 1)
              ).astype(jnp.float32)
    out_ref[:, :64] = jnp.dot(onehot, tbl, preferred_element_type=jnp.float32)
    out_ref[:, 64:] = jnp.zeros((BN, 64), jnp.float32)


def _mlp_body(x_ref, agg_ref, w1_ref, b1_ref, w2_ref, b2_ref, out_ref):
    u = x_ref[:, :64] + agg_ref[:, :64]
    t = jnp.maximum(
        jnp.dot(u, w1_ref[...], preferred_element_type=jnp.float32)
        + b1_ref[...], 0.0)
    h = jnp.dot(t, w2_ref[...], preferred_element_type=jnp.float32) + b2_ref[...]
    out_ref[:, :64] = jnp.maximum(h, 0.0)
    out_ref[:, 64:] = jnp.zeros((BN, 64), jnp.float32)


def _final_body(x_ref, agg_ref, w1_ref, b1_ref, w2_ref, b2_ref, wf_ref, bf_ref,
                out_ref):
    u = x_ref[:, :64] + agg_ref[:, :64]
    t = jnp.maximum(
        jnp.dot(u, w1_ref[...], preferred_element_type=jnp.float32)
        + b1_ref[...], 0.0)
    h = jnp.dot(t, w2_ref[...], preferred_element_type=jnp.float32) + b2_ref[...]
    y = jnp.dot(h, wf_ref[...], preferred_element_type=jnp.float32) + bf_ref[...]
    out_ref[:, :64] = y
    ext = (lax.broadcasted_iota(jnp.int32, (BN, 32), 1) == 0).astype(jnp.float32)
    out_ref[:, 64:] = ext


def _divide_body(p_ref, out_ref):
    p = p_ref[...]
    s = p[0] + p[1]
    cnt = jnp.maximum(s[:G, 64:65], 1.0)
    out_ref[...] = s[:G, :64] / cnt


_MESH = plsc.VectorSubcoreMesh(core_axis_name="c", subcore_axis_name="s",
                               num_cores=2, num_subcores=16)


def _edge_sc_body(x_hbm, src_hbm, dst_hbm, attr_hbm, w_hbm, agg_hbm,
                  agg_sp, wv, srcv, dstv, attrv, dstl, xrows, mrows, zbuf,
                  sem_i, sem_g, sem_s):
    c = lax.axis_index("c")
    s = lax.axis_index("s")
    cb = c * 32
    pltpu.sync_copy(w_hbm, wv)
    wk = [[wv[k, pl.ds(cb + g * 16, 16)] for g in range(2)] for k in range(4)]
    bk = [wv[4, pl.ds(cb + g * 16, 16)] for g in range(2)]

    z = jnp.zeros((16,), jnp.float32)

    def zrow(i, carry):
        for g in range(2):
            zbuf[i, pl.ds(g * 16, 16)] = z
        return carry

    lax.fori_loop(0, 16, zrow, 0)

    def zcopy(k, carry):
        pltpu.sync_copy(zbuf, agg_sp.at[pl.ds(s * TPN + k * 16, 16)])
        return carry

    lax.fori_loop(0, TPN // 16, zcopy, 0)
    plsc.subcore_barrier()

    nk = (NCHT - s + 15) // 16

    def ins_in(i, b):
        e0 = (s + 16 * i) * CH
        pltpu.async_copy(src_hbm.at[pl.ds(e0, CH)], srcv.at[b], sem_i.at[b])
        pltpu.async_copy(dst_hbm.at[pl.ds(e0, CH)], dstv.at[b], sem_i.at[b])
        pltpu.async_copy(attr_hbm.at[pl.ds(e0 * 4, CH * 4)], attrv.at[b],
                         sem_i.at[b])

    def ins_wait(i, b):
        e0 = (s + 16 * i) * CH
        pltpu.make_async_copy(src_hbm.at[pl.ds(e0, CH)], srcv.at[b],
                              sem_i.at[b]).wait()
        pltpu.make_async_copy(dst_hbm.at[pl.ds(e0, CH)], dstv.at[b],
                              sem_i.at[b]).wait()
        pltpu.make_async_copy(attr_hbm.at[pl.ds(e0 * 4, CH * 4)], attrv.at[b],
                              sem_i.at[b]).wait()

    def gather(b):
        pltpu.async_copy(x_hbm.at[srcv.at[b]], xrows.at[b], sem_g.at[b])

    def gather_wait(b):
        pltpu.make_async_copy(x_hbm.at[srcv.at[b]], xrows.at[b],
                              sem_g.at[b]).wait()

    def scatter(b):
        pltpu.async_copy(mrows.at[b], agg_sp.at[dstl.at[b]],
                         sem_s.at[b], add=True)

    def scatter_wait(b):
        pltpu.make_async_copy(mrows.at[b], agg_sp.at[dstl.at[b]],
                              sem_s.at[b]).wait()

    ins_in(0, 0)
    ins_in(1, 1)
    ins_wait(0, 0)
    gather(0)

    def outer(io, carry):
        for b in range(2):
            i = io * 2 + b
            nb = 1 - b

            @pl.when(i < nk)
            def _process():
                @pl.when(i >= 2)
                def _():
                    scatter_wait(b)

                gather_wait(b)

                @pl.when(i + 1 < nk)
                def _():
                    ins_wait(i + 1, nb)
                    gather(nb)

                def grp(q, ecarry):
                    av = lax.bitcast_convert_type(
                        attrv[b, pl.ds(q * 16, 16)], jnp.float32)
                    for t in range(4):
                        j = q * 4 + t
                        a = [av[4 * t + k] for k in range(4)]
                        for g in range(2):
                            e = (bk[g] + a[0] * wk[0][g] + a[1] * wk[1][g]
                                 + a[2] * wk[2][g] + a[3] * wk[3][g])
                            m = jnp.maximum(
                                xrows[b, j, pl.ds(cb + g * 16, 16)] + e, 0.0)
                            mrows[b, j, pl.ds(g * 16, 16)] = m
                    return ecarry

                lax.fori_loop(0, CH // 4, grp, 0)

                for v in range(CH // 16):
                    dstl[b, pl.ds(v * 16, 16)] = dstv[b, pl.ds(v * 16, 16)]

                scatter(b)

                @pl.when(i + 2 < nk)
                def _():
                    ins_in(i + 2, b)

        return carry

    lax.fori_loop(0, (nk + 2) // 2, outer, 0)
    scatter_wait(1)
    scatter_wait(0)
    plsc.subcore_barrier()

    @pl.when(c == 0)
    def _out0():
        pltpu.sync_copy(agg_sp.at[pl.ds(s * TPN, TPN)],
                        agg_hbm.at[pl.ds(s * TPN, TPN), pl.ds(0, 32)])

    @pl.when(c == 1)
    def _out1():
        pltpu.sync_copy(agg_sp.at[pl.ds(s * TPN, TPN)],
                        agg_hbm.at[pl.ds(s * TPN, TPN), pl.ds(32, 32)])


_edge_sc = pl.kernel(
    _edge_sc_body,
    out_type=jax.ShapeDtypeStruct((NP, 128), jnp.float32),
    mesh=_MESH,
    compiler_params=pltpu.CompilerParams(use_tc_tiling_on_sc=False),
    scratch_types=[
        pltpu.VMEM_SHARED((AROWS, 32), jnp.float32),
        pltpu.VMEM((5, HID), jnp.float32),
        pltpu.VMEM((2, CH), jnp.int32),
        pltpu.VMEM((2, CH), jnp.int32),
        pltpu.VMEM((2, 4 * CH), jnp.int32),
        pltpu.VMEM((2, CH), jnp.int32),
        pltpu.VMEM((2, CH, 128), jnp.float32),
        pltpu.VMEM((2, CH, 32), jnp.float32),
        pltpu.VMEM((16, 32), jnp.float32),
        pltpu.SemaphoreType.DMA((2,)),
        pltpu.SemaphoreType.DMA((2,)),
        pltpu.SemaphoreType.DMA((2,)),
    ],
)


def _pool_sc_body(y_hbm, batch_hbm, out_hbm, pool_sp, rows_v, bidv, zbuf):
    c = lax.axis_index("c")
    s = lax.axis_index("s")
    w = c * 16 + s

    z = jnp.zeros((16,), jnp.float32)

    def zrow(i, carry):
        for g in range(6):
            zbuf[i, pl.ds(g * 16, 16)] = z
        return carry

    lax.fori_loop(0, 65, zrow, 0)
    pltpu.sync_copy(zbuf, pool_sp.at[pl.ds(s * 65, 65)])
    plsc.subcore_barrier()

    def chunk(k, carry):
        base = (w + k * 32) * PCH
        pltpu.sync_copy(y_hbm.at[pl.ds(base, PCH)], rows_v)
        pltpu.sync_copy(batch_hbm.at[pl.ds(base, PCH)], bidv)
        pltpu.sync_copy(rows_v, pool_sp.at[bidv], add=True)
        return carry

    lax.fori_loop(0, PNW, chunk, 0)
    plsc.subcore_barrier()
    pltpu.sync_copy(pool_sp.at[pl.ds(s * 65, 65)],
                    out_hbm.at[c, pl.ds(s * 65, 65)])


_pool_sc = pl.kernel(
    _pool_sc_body,
    out_type=jax.ShapeDtypeStruct((2, GP, 96), jnp.float32),
    mesh=_MESH,
    compiler_params=pltpu.CompilerParams(use_tc_tiling_on_sc=False),
    scratch_types=[
        pltpu.VMEM_SHARED((GP, 96), jnp.float32),
        pltpu.VMEM((PCH, 96), jnp.float32),
        pltpu.VMEM((PCH,), jnp.int32),
        pltpu.VMEM((65, 96), jnp.float32),
    ],
)


def _agg_spec():
    return pl.BlockSpec((BN, 128), lambda i: (i, 0))


def _full(shape):
    return pl.BlockSpec(shape, lambda i: tuple(0 for _ in shape))


def kernel(card_ids, edge_index, edge_attr, batch, emb, Wp, bp, W1, b1, W2, b2,
           We1, be1, We2, be2, Wf, bf):
    pad = NP - N
    ids_pad = jnp.concatenate(
        [card_ids, jnp.full((pad,), 52, card_ids.dtype)])
    ids3 = ids_pad.reshape(NB, 1, BN)
    batch_pad = jnp.concatenate(
        [batch, (G + jnp.arange(pad, dtype=batch.dtype) % 16)])
    emb_pad = jnp.zeros((64, 16), jnp.float32).at[:53, :].set(emb)
    b1r = b1.reshape(1, HID)
    b2r = b2.reshape(1, HID)
    bpr = bp.reshape(1, HID)
    bfr = bf.reshape(1, HID)
    wrow1 = jnp.concatenate([We1, be1[None, :]], axis=0)
    wrow2 = jnp.concatenate([We2, be2[None, :]], axis=0)

    x0 = pl.pallas_call(
        _embed_body,
        grid=(NB,),
        in_specs=[
            pl.BlockSpec((1, 1, BN), lambda i: (i, 0, 0)),
            _full((64, 16)),
            _full((16, HID)),
            _full((1, HID)),
        ],
        out_specs=pl.BlockSpec((BN, 128), lambda i: (i, 0)),
        out_shape=jax.ShapeDtypeStruct((NP, 128), jnp.float32),
    )(ids3, emb_pad, Wp, bpr)

    src = edge_index[0]
    dst = edge_index[1]
    attr_flat = lax.bitcast_convert_type(edge_attr, jnp.int32).reshape(E * 4)

    agg1 = _edge_sc(x0, src, dst, attr_flat, wrow1)

    h1 = pl.pallas_call(
        _mlp_body,
        grid=(NB,),
        in_specs=[
            pl.BlockSpec((BN, 128), lambda i: (i, 0)),
            _agg_spec(),
            _full((HID, HID)),
            _full((1, HID)),
            _full((HID, HID)),
            _full((1, HID)),
        ],
        out_specs=pl.BlockSpec((BN, 128), lambda i: (i, 0)),
        out_shape=jax.ShapeDtypeStruct((NP, 128), jnp.float32),
    )(x0, agg1, W1, b1r, W2, b2r)

    agg2 = _edge_sc(h1, src, dst, attr_flat, wrow2)

    yext = pl.pallas_call(
        _final_body,
        grid=(NB,),
        in_specs=[
            pl.BlockSpec((BN, 128), lambda i: (i, 0)),
            _agg_spec(),
            _full((HID, HID)),
            _full((1, HID)),
            _full((HID, HID)),
            _full((1, HID)),
            _full((HID, HID)),
            _full((1, HID)),
        ],
        out_specs=pl.BlockSpec((BN, 96), lambda i: (i, 0)),
        out_shape=jax.ShapeDtypeStruct((NP, 96), jnp.float32),
    )(h1, agg2, W1, b1r, W2, b2r, Wf, bfr)

    pooled = _pool_sc(yext, batch_pad)

    out = pl.pallas_call(
        _divide_body,
        in_specs=[pl.BlockSpec((2, GP, 96), lambda: (0, 0, 0))],
        out_specs=pl.BlockSpec((G, HID), lambda: (0, 0)),
        out_shape=jax.ShapeDtypeStruct((G, HID), jnp.float32),
    )(pooled)
    return out
